# Initial kernel scaffold; baseline (speedup 1.0000x reference)
import jax
import jax.numpy as jnp
from jax.experimental import pallas as pl


def kernel(users, items, adj_entity, adj_relation, user_table, entity_table, relation_table, W_agg, b_agg):
    raise NotImplementedError("write your pallas kernel here")


# speedup vs baseline: 4.4390x; 4.4390x over previous
import functools

import jax
import jax.numpy as jnp
from jax import lax
from jax.experimental import pallas as pl
from jax.experimental.pallas import tpu as pltpu
from jax.experimental.pallas import tpu_sc as plsc

E_DIM = 64
N_NEIGHBORS = 16
B = 16384

NC = 2
NS = 16
NW = NC * NS
PER_W = B // NW
CHUNK = 64
N_CHUNKS = PER_W // CHUNK
IDX_PER_CHUNK = CHUNK * N_NEIGHBORS
STREAM = 128
N_STREAMS = IDX_PER_CHUNK // STREAM


def _norm_body(x_ref, o_ref):
    x = x_ref[...]
    n = jnp.sqrt(jnp.sum(x * x, axis=-1, keepdims=True))
    scale = jnp.minimum(1.0, 1.0 / jnp.maximum(n, 1e-7))
    o_ref[...] = x * scale


def _normalize_table(table):
    rows = table.shape[0]
    blk = 2000
    grid = rows // blk
    return pl.pallas_call(
        _norm_body,
        grid=(grid,),
        in_specs=[pl.BlockSpec((blk, E_DIM), lambda i: (i, 0))],
        out_specs=pl.BlockSpec((blk, E_DIM), lambda i: (i, 0)),
        out_shape=jax.ShapeDtypeStruct((rows, E_DIM), jnp.float32),
    )(table)


def _sc_body(users_hbm, items_hbm, adj_e_hbm, adj_r_hbm, user_tab_hbm,
             ent_norm_hbm, out_u, out_e0, out_nbr, out_rel,
             idx_u, idx_i, nbr_ids, nbr_flat, rel_buf, rows_u, rows_e0,
             nbr_rows, sem_ids, sem_rel, sem_rows, sem_nbr):
    wid = lax.axis_index("s") * NC + lax.axis_index("c")

    def chunk_body(t, carry):
        base = wid * PER_W + t * CHUNK
        pltpu.sync_copy(users_hbm.at[pl.ds(base, CHUNK)], idx_u)
        pltpu.sync_copy(items_hbm.at[pl.ds(base, CHUNK)], idx_i)

        h_ids = pltpu.async_copy(adj_e_hbm.at[idx_i], nbr_ids, sem_ids)
        h_rel = pltpu.async_copy(adj_r_hbm.at[idx_i], rel_buf, sem_rel)
        h_u = pltpu.async_copy(user_tab_hbm.at[idx_u], rows_u, sem_rows)
        h_e0 = pltpu.async_copy(ent_norm_hbm.at[idx_i], rows_e0, sem_rows)

        h_ids.wait()

        def flat_body(c, _):
            v = nbr_ids[c, :]
            nbr_flat[pl.ds(pl.multiple_of(c * 16, 16), 16)] = v
            return 0

        lax.fori_loop(0, CHUNK, flat_body, 0)

        h_nbr = []
        for j in range(N_STREAMS):
            sl = pl.ds(j * STREAM, STREAM)
            h_nbr.append(pltpu.async_copy(
                ent_norm_hbm.at[nbr_flat.at[sl]], nbr_rows.at[sl], sem_nbr))
        h_u.wait()
        h_e0.wait()
        h_rel.wait()
        for h in h_nbr:
            h.wait()

        pltpu.sync_copy(rows_u, out_u.at[pl.ds(base, CHUNK)])
        pltpu.sync_copy(rows_e0, out_e0.at[pl.ds(base, CHUNK)])
        pltpu.sync_copy(nbr_rows, out_nbr.at[pl.ds(base * 16, IDX_PER_CHUNK)])
        pltpu.sync_copy(rel_buf, out_rel.at[pl.ds(base, CHUNK)])
        return carry

    lax.fori_loop(0, N_CHUNKS, chunk_body, 0)


def _sc_gather(users_i, items_i, adj_e, adj_r, user_table, ent_norm):
    mesh = plsc.VectorSubcoreMesh(core_axis_name="c", subcore_axis_name="s")
    f = functools.partial(
        pl.kernel,
        out_type=(
            jax.ShapeDtypeStruct((B, E_DIM), jnp.float32),
            jax.ShapeDtypeStruct((B, E_DIM), jnp.float32),
            jax.ShapeDtypeStruct((B * N_NEIGHBORS, E_DIM), jnp.float32),
            jax.ShapeDtypeStruct((B, N_NEIGHBORS), jnp.int32),
        ),
        mesh=mesh,
        compiler_params=pltpu.CompilerParams(use_tc_tiling_on_sc=False),
        scratch_types=(
            pltpu.VMEM((CHUNK,), jnp.int32),
            pltpu.VMEM((CHUNK,), jnp.int32),
            pltpu.VMEM((CHUNK, N_NEIGHBORS), jnp.int32),
            pltpu.VMEM((IDX_PER_CHUNK,), jnp.int32),
            pltpu.VMEM((CHUNK, N_NEIGHBORS), jnp.int32),
            pltpu.VMEM((CHUNK, E_DIM), jnp.float32),
            pltpu.VMEM((CHUNK, E_DIM), jnp.float32),
            pltpu.VMEM((IDX_PER_CHUNK, E_DIM), jnp.float32),
            pltpu.SemaphoreType.DMA,
            pltpu.SemaphoreType.DMA,
            pltpu.SemaphoreType.DMA,
            pltpu.SemaphoreType.DMA,
        ),
    )(_sc_body)
    return f(users_i, items_i, adj_e, adj_r, user_table, ent_norm)


def _combine_body(u_ref, e0_ref, nbr_ref, rel_ref, reltab_ref, w_ref, b_ref,
                  uout_ref, iout_ref):
    rel = reltab_ref[...]
    rn = jnp.sqrt(jnp.sum(rel * rel, axis=-1, keepdims=True))
    rel_n = rel * jnp.minimum(1.0, 1.0 / jnp.maximum(rn, 1e-7))

    u = u_ref[...]
    un = jnp.sqrt(jnp.sum(u * u, axis=-1, keepdims=True))
    u_n = u * jnp.minimum(1.0, 1.0 / jnp.maximum(un, 1e-7))
    uout_ref[...] = u_n

    p = lax.dot_general(u_n, rel_n, (((1,), (1,)), ((), ())))
    ids = rel_ref[...]
    s = jnp.zeros(ids.shape, jnp.float32)
    for r in range(32):
        s = s + jnp.where(ids == r, p[:, r:r + 1], 0.0)

    m = jnp.max(s, axis=1, keepdims=True)
    e = jnp.exp(s - m)
    w = e / jnp.sum(e, axis=1, keepdims=True)

    agg = jnp.zeros_like(u)
    for k in range(N_NEIGHBORS):
        agg = agg + w[:, k:k + 1] * nbr_ref[:, k, :]

    out = (e0_ref[...] + agg) @ w_ref[...] + b_ref[...]
    iout_ref[...] = jnp.tanh(out)


def _tc_combine(rows_u, rows_e0, nbr3, rel2, relation_table, W_agg, b2):
    bt = 256
    grid = B // bt
    return pl.pallas_call(
        _combine_body,
        grid=(grid,),
        in_specs=[
            pl.BlockSpec((bt, E_DIM), lambda i: (i, 0)),
            pl.BlockSpec((bt, E_DIM), lambda i: (i, 0)),
            pl.BlockSpec((bt, N_NEIGHBORS, E_DIM), lambda i: (i, 0, 0)),
            pl.BlockSpec((bt, N_NEIGHBORS), lambda i: (i, 0)),
            pl.BlockSpec((32, E_DIM), lambda i: (0, 0)),
            pl.BlockSpec((E_DIM, E_DIM), lambda i: (0, 0)),
            pl.BlockSpec((1, E_DIM), lambda i: (0, 0)),
        ],
        out_specs=[
            pl.BlockSpec((bt, E_DIM), lambda i: (i, 0)),
            pl.BlockSpec((bt, E_DIM), lambda i: (i, 0)),
        ],
        out_shape=[
            jax.ShapeDtypeStruct((B, E_DIM), jnp.float32),
            jax.ShapeDtypeStruct((B, E_DIM), jnp.float32),
        ],
    )(rows_u, rows_e0, nbr3, rel2, relation_table, W_agg, b2)


def kernel(users, items, adj_entity, adj_relation, user_table, entity_table,
           relation_table, W_agg, b_agg):
    users_i = users.astype(jnp.int32)
    items_i = items.astype(jnp.int32)
    adj_e = adj_entity.astype(jnp.int32)
    adj_r = adj_relation.astype(jnp.int32)

    ent_norm = _normalize_table(entity_table)
    rows_u, rows_e0, nbr_rows, rel2 = _sc_gather(
        users_i, items_i, adj_e, adj_r, user_table, ent_norm)

    nbr3 = nbr_rows.reshape(B, N_NEIGHBORS, E_DIM)
    user_emb, item_out = _tc_combine(
        rows_u, rows_e0, nbr3, rel2, relation_table, W_agg,
        b_agg.reshape(1, E_DIM))
    return user_emb.reshape(B, 1, E_DIM), item_out


# speedup vs baseline: 5.7910x; 1.3046x over previous
import functools

import jax
import jax.numpy as jnp
from jax import lax
from jax.experimental import pallas as pl
from jax.experimental.pallas import tpu as pltpu
from jax.experimental.pallas import tpu_sc as plsc

E_DIM = 64
N_NEIGHBORS = 16
B = 16384

NC = 2
NS = 16
NW = NC * NS
PER_W = B // NW

CHUNK1 = 64
N_CHUNKS1 = PER_W // CHUNK1
IDX1 = CHUNK1 * N_NEIGHBORS
CHUNK2 = 64
N_CHUNKS2 = PER_W // CHUNK2
IDX2 = CHUNK2 * N_NEIGHBORS
STREAM = 128
N_STREAMS2 = IDX2 // STREAM


def _norm_body(x_ref, o_ref):
    x = x_ref[...]
    n = jnp.sqrt(jnp.sum(x * x, axis=-1, keepdims=True))
    scale = jnp.minimum(1.0, 1.0 / jnp.maximum(n, 1e-7))
    o_ref[...] = x * scale


def _normalize_table(table):
    rows = table.shape[0]
    blk = 2000
    grid = rows // blk
    return pl.pallas_call(
        _norm_body,
        grid=(grid,),
        in_specs=[pl.BlockSpec((blk, E_DIM), lambda i: (i, 0))],
        out_specs=pl.BlockSpec((blk, E_DIM), lambda i: (i, 0)),
        out_shape=jax.ShapeDtypeStruct((rows, E_DIM), jnp.float32),
    )(table)


def _sc1_body(users_hbm, items_hbm, adj_e_hbm, adj_r_hbm, user_tab_hbm,
              ent_tab_hbm, out_u, out_e0, out_nid, out_rel,
              idx_u, idx_i, nbr_ids, nbr_flat, rel_buf, rows_u, rows_e0,
              sem_a, sem_b):
    wid = lax.axis_index("s") * NC + lax.axis_index("c")

    def chunk_body(t, carry):
        base = wid * PER_W + t * CHUNK1
        pltpu.sync_copy(users_hbm.at[pl.ds(base, CHUNK1)], idx_u)
        pltpu.sync_copy(items_hbm.at[pl.ds(base, CHUNK1)], idx_i)

        h_ids = pltpu.async_copy(adj_e_hbm.at[idx_i], nbr_ids, sem_a)
        h_rel = pltpu.async_copy(adj_r_hbm.at[idx_i], rel_buf, sem_b)
        h_u = pltpu.async_copy(user_tab_hbm.at[idx_u], rows_u, sem_b)
        h_e0 = pltpu.async_copy(ent_tab_hbm.at[idx_i], rows_e0, sem_b)

        h_ids.wait()

        def flat_body(c, _):
            v = nbr_ids[c, :]
            nbr_flat[pl.ds(pl.multiple_of(c * 16, 16), 16)] = v
            return 0

        lax.fori_loop(0, CHUNK1, flat_body, 0)

        h_rel.wait()
        h_u.wait()
        h_e0.wait()

        pltpu.sync_copy(rows_u, out_u.at[pl.ds(base, CHUNK1)])
        pltpu.sync_copy(rows_e0, out_e0.at[pl.ds(base, CHUNK1)])
        pltpu.sync_copy(nbr_flat, out_nid.at[pl.ds(base * 16, IDX1)])
        pltpu.sync_copy(rel_buf, out_rel.at[pl.ds(base, CHUNK1)])
        return carry

    lax.fori_loop(0, N_CHUNKS1, chunk_body, 0)


def _sc1_gather(users_i, items_i, adj_e, adj_r, user_table, entity_table):
    mesh = plsc.VectorSubcoreMesh(core_axis_name="c", subcore_axis_name="s")
    f = functools.partial(
        pl.kernel,
        out_type=(
            jax.ShapeDtypeStruct((B, E_DIM), jnp.float32),
            jax.ShapeDtypeStruct((B, E_DIM), jnp.float32),
            jax.ShapeDtypeStruct((B * N_NEIGHBORS,), jnp.int32),
            jax.ShapeDtypeStruct((B, N_NEIGHBORS), jnp.int32),
        ),
        mesh=mesh,
        compiler_params=pltpu.CompilerParams(use_tc_tiling_on_sc=False),
        scratch_types=(
            pltpu.VMEM((CHUNK1,), jnp.int32),
            pltpu.VMEM((CHUNK1,), jnp.int32),
            pltpu.VMEM((CHUNK1, N_NEIGHBORS), jnp.int32),
            pltpu.VMEM((IDX1,), jnp.int32),
            pltpu.VMEM((CHUNK1, N_NEIGHBORS), jnp.int32),
            pltpu.VMEM((CHUNK1, E_DIM), jnp.float32),
            pltpu.VMEM((CHUNK1, E_DIM), jnp.float32),
            pltpu.SemaphoreType.DMA,
            pltpu.SemaphoreType.DMA,
        ),
    )(_sc1_body)
    return f(users_i, items_i, adj_e, adj_r, user_table, entity_table)


def _weights_body(u_ref, e0_ref, rel_ref, reltab_ref, uout_ref, e0out_ref,
                  w_ref):
    rel = reltab_ref[...]
    rn = jnp.sqrt(jnp.sum(rel * rel, axis=-1, keepdims=True))
    rel_n = rel * jnp.minimum(1.0, 1.0 / jnp.maximum(rn, 1e-7))

    u = u_ref[...]
    un = jnp.sqrt(jnp.sum(u * u, axis=-1, keepdims=True))
    u_n = u * jnp.minimum(1.0, 1.0 / jnp.maximum(un, 1e-7))
    uout_ref[...] = u_n

    e0 = e0_ref[...]
    en = jnp.sqrt(jnp.sum(e0 * e0, axis=-1, keepdims=True))
    e0out_ref[...] = e0 * jnp.minimum(1.0, 1.0 / jnp.maximum(en, 1e-7))

    p = lax.dot_general(u_n, rel_n, (((1,), (1,)), ((), ())))
    ids = rel_ref[...]
    s = jnp.zeros(ids.shape, jnp.float32)
    for r in range(32):
        s = s + jnp.where(ids == r, p[:, r:r + 1], 0.0)

    m = jnp.max(s, axis=1, keepdims=True)
    e = jnp.exp(s - m)
    w_ref[...] = e / jnp.sum(e, axis=1, keepdims=True)


def _tc_weights(rows_u, rows_e0, rel2, relation_table):
    bt = 512
    grid = B // bt
    return pl.pallas_call(
        _weights_body,
        grid=(grid,),
        in_specs=[
            pl.BlockSpec((bt, E_DIM), lambda i: (i, 0)),
            pl.BlockSpec((bt, E_DIM), lambda i: (i, 0)),
            pl.BlockSpec((bt, N_NEIGHBORS), lambda i: (i, 0)),
            pl.BlockSpec((32, E_DIM), lambda i: (0, 0)),
        ],
        out_specs=[
            pl.BlockSpec((bt, E_DIM), lambda i: (i, 0)),
            pl.BlockSpec((bt, E_DIM), lambda i: (i, 0)),
            pl.BlockSpec((bt, N_NEIGHBORS), lambda i: (i, 0)),
        ],
        out_shape=[
            jax.ShapeDtypeStruct((B, E_DIM), jnp.float32),
            jax.ShapeDtypeStruct((B, E_DIM), jnp.float32),
            jax.ShapeDtypeStruct((B, N_NEIGHBORS), jnp.float32),
        ],
    )(rows_u, rows_e0, rel2, relation_table)


def _sc2_body(nid_hbm, w_hbm, ent_norm_hbm, out_agg,
              ids_v, w_v, rows_v, agg_v, sem_a, sem_b):
    wid = lax.axis_index("s") * NC + lax.axis_index("c")

    def chunk_body(t, carry):
        base = wid * PER_W + t * CHUNK2
        h_ids = pltpu.async_copy(
            nid_hbm.at[pl.ds(base * 16, IDX2)], ids_v, sem_a)
        h_w = pltpu.async_copy(w_hbm.at[pl.ds(base * 16, IDX2)], w_v, sem_b)
        h_ids.wait()
        h_rows = []
        for j in range(N_STREAMS2):
            sl = pl.ds(j * STREAM, STREAM)
            h_rows.append(pltpu.async_copy(
                ent_norm_hbm.at[ids_v.at[sl]], rows_v.at[sl], sem_a))
        h_w.wait()
        for h in h_rows:
            h.wait()

        def elem_body(c, _):
            wv = w_v[pl.ds(pl.multiple_of(c * 16, 16), 16)]
            for g in range(E_DIM // 16):
                acc = jnp.zeros((16,), jnp.float32)
                for k in range(N_NEIGHBORS):
                    acc = acc + wv[k] * rows_v[c * 16 + k,
                                               pl.ds(g * 16, 16)]
                agg_v[c, pl.ds(g * 16, 16)] = acc
            return 0

        lax.fori_loop(0, CHUNK2, elem_body, 0)

        pltpu.sync_copy(agg_v, out_agg.at[pl.ds(base, CHUNK2)])
        return carry

    lax.fori_loop(0, N_CHUNKS2, chunk_body, 0)


def _sc2_reduce(nbr_ids_flat, w_flat, ent_norm):
    mesh = plsc.VectorSubcoreMesh(core_axis_name="c", subcore_axis_name="s")
    f = functools.partial(
        pl.kernel,
        out_type=jax.ShapeDtypeStruct((B, E_DIM), jnp.float32),
        mesh=mesh,
        compiler_params=pltpu.CompilerParams(use_tc_tiling_on_sc=False),
        scratch_types=(
            pltpu.VMEM((IDX2,), jnp.int32),
            pltpu.VMEM((IDX2,), jnp.float32),
            pltpu.VMEM((IDX2, E_DIM), jnp.float32),
            pltpu.VMEM((CHUNK2, E_DIM), jnp.float32),
            pltpu.SemaphoreType.DMA,
            pltpu.SemaphoreType.DMA,
        ),
    )(_sc2_body)
    return f(nbr_ids_flat, w_flat, ent_norm)


def _final_body(e0_ref, agg_ref, w_ref, b_ref, o_ref):
    out = (e0_ref[...] + agg_ref[...]) @ w_ref[...] + b_ref[...]
    o_ref[...] = jnp.tanh(out)


def _tc_final(e0_n, agg, W_agg, b2):
    bt = 1024
    grid = B // bt
    return pl.pallas_call(
        _final_body,
        grid=(grid,),
        in_specs=[
            pl.BlockSpec((bt, E_DIM), lambda i: (i, 0)),
            pl.BlockSpec((bt, E_DIM), lambda i: (i, 0)),
            pl.BlockSpec((E_DIM, E_DIM), lambda i: (0, 0)),
            pl.BlockSpec((1, E_DIM), lambda i: (0, 0)),
        ],
        out_specs=pl.BlockSpec((bt, E_DIM), lambda i: (i, 0)),
        out_shape=jax.ShapeDtypeStruct((B, E_DIM), jnp.float32),
    )(e0_n, agg, W_agg, b2)


def kernel(users, items, adj_entity, adj_relation, user_table, entity_table,
           relation_table, W_agg, b_agg):
    users_i = users.astype(jnp.int32)
    items_i = items.astype(jnp.int32)
    adj_e = adj_entity.astype(jnp.int32)
    adj_r = adj_relation.astype(jnp.int32)

    ent_norm = _normalize_table(entity_table)
    rows_u, rows_e0, nbr_ids_flat, rel2 = _sc1_gather(
        users_i, items_i, adj_e, adj_r, user_table, entity_table)
    u_n, e0_n, w = _tc_weights(rows_u, rows_e0, rel2, relation_table)
    agg = _sc2_reduce(nbr_ids_flat, w.reshape(-1), ent_norm)
    item_out = _tc_final(e0_n, agg, W_agg, b_agg.reshape(1, E_DIM))
    return u_n.reshape(B, 1, E_DIM), item_out


# speedup vs baseline: 5.9520x; 1.0278x over previous
import functools

import jax
import jax.numpy as jnp
from jax import lax
from jax.experimental import pallas as pl
from jax.experimental.pallas import tpu as pltpu
from jax.experimental.pallas import tpu_sc as plsc

E_DIM = 64
N_NEIGHBORS = 16
B = 16384

NC = 2
NS = 16
NW = NC * NS
PER_W = B // NW

CHUNK1 = 64
N_CHUNKS1 = PER_W // CHUNK1
IDX1 = CHUNK1 * N_NEIGHBORS
CHUNK2 = 64
N_CHUNKS2 = PER_W // CHUNK2
IDX2 = CHUNK2 * N_NEIGHBORS
STREAM = 128
N_STREAMS2 = IDX2 // STREAM


def _norm_body(x_ref, o_ref):
    x = x_ref[...]
    n = jnp.sqrt(jnp.sum(x * x, axis=-1, keepdims=True))
    o_ref[...] = jnp.minimum(1.0, 1.0 / jnp.maximum(n, 1e-7))


def _row_scales(table):
    rows = table.shape[0]
    blk = 2000
    grid = rows // blk
    return pl.pallas_call(
        _norm_body,
        grid=(grid,),
        in_specs=[pl.BlockSpec((blk, E_DIM), lambda i: (i, 0))],
        out_specs=pl.BlockSpec((blk, 1), lambda i: (i, 0)),
        out_shape=jax.ShapeDtypeStruct((rows, 1), jnp.float32),
    )(table)


def _sc1_body(users_hbm, items_hbm, adj_e_hbm, adj_r_hbm, user_tab_hbm,
              ent_tab_hbm, out_u, out_e0, out_nid, out_rel,
              idx_u, idx_i, nbr_ids, nbr_flat, rel_buf, rows_u, rows_e0,
              sem_a, sem_b):
    wid = lax.axis_index("s") * NC + lax.axis_index("c")

    def chunk_body(t, carry):
        base = wid * PER_W + t * CHUNK1
        pltpu.sync_copy(users_hbm.at[pl.ds(base, CHUNK1)], idx_u)
        pltpu.sync_copy(items_hbm.at[pl.ds(base, CHUNK1)], idx_i)

        h_ids = pltpu.async_copy(adj_e_hbm.at[idx_i], nbr_ids, sem_a)
        h_rel = pltpu.async_copy(adj_r_hbm.at[idx_i], rel_buf, sem_b)
        h_u = pltpu.async_copy(user_tab_hbm.at[idx_u], rows_u, sem_b)
        h_e0 = pltpu.async_copy(ent_tab_hbm.at[idx_i], rows_e0, sem_b)

        h_ids.wait()

        def flat_body(c, _):
            v = nbr_ids[c, :]
            nbr_flat[pl.ds(pl.multiple_of(c * 16, 16), 16)] = v
            return 0

        lax.fori_loop(0, CHUNK1, flat_body, 0)

        h_rel.wait()
        h_u.wait()
        h_e0.wait()

        pltpu.sync_copy(rows_u, out_u.at[pl.ds(base, CHUNK1)])
        pltpu.sync_copy(rows_e0, out_e0.at[pl.ds(base, CHUNK1)])
        pltpu.sync_copy(nbr_flat, out_nid.at[pl.ds(base * 16, IDX1)])
        pltpu.sync_copy(rel_buf, out_rel.at[pl.ds(base, CHUNK1)])
        return carry

    lax.fori_loop(0, N_CHUNKS1, chunk_body, 0)


def _sc1_gather(users_i, items_i, adj_e, adj_r, user_table, entity_table):
    mesh = plsc.VectorSubcoreMesh(core_axis_name="c", subcore_axis_name="s")
    f = functools.partial(
        pl.kernel,
        out_type=(
            jax.ShapeDtypeStruct((B, E_DIM), jnp.float32),
            jax.ShapeDtypeStruct((B, E_DIM), jnp.float32),
            jax.ShapeDtypeStruct((B * N_NEIGHBORS,), jnp.int32),
            jax.ShapeDtypeStruct((B, N_NEIGHBORS), jnp.int32),
        ),
        mesh=mesh,
        compiler_params=pltpu.CompilerParams(use_tc_tiling_on_sc=False),
        scratch_types=(
            pltpu.VMEM((CHUNK1,), jnp.int32),
            pltpu.VMEM((CHUNK1,), jnp.int32),
            pltpu.VMEM((CHUNK1, N_NEIGHBORS), jnp.int32),
            pltpu.VMEM((IDX1,), jnp.int32),
            pltpu.VMEM((CHUNK1, N_NEIGHBORS), jnp.int32),
            pltpu.VMEM((CHUNK1, E_DIM), jnp.float32),
            pltpu.VMEM((CHUNK1, E_DIM), jnp.float32),
            pltpu.SemaphoreType.DMA,
            pltpu.SemaphoreType.DMA,
        ),
    )(_sc1_body)
    return f(users_i, items_i, adj_e, adj_r, user_table, entity_table)


def _weights_body(u_ref, e0_ref, rel_ref, reltab_ref, uout_ref, e0out_ref,
                  w_ref):
    rel = reltab_ref[...]
    rn = jnp.sqrt(jnp.sum(rel * rel, axis=-1, keepdims=True))
    rel_n = rel * jnp.minimum(1.0, 1.0 / jnp.maximum(rn, 1e-7))

    u = u_ref[...]
    un = jnp.sqrt(jnp.sum(u * u, axis=-1, keepdims=True))
    u_n = u * jnp.minimum(1.0, 1.0 / jnp.maximum(un, 1e-7))
    uout_ref[...] = u_n

    e0 = e0_ref[...]
    en = jnp.sqrt(jnp.sum(e0 * e0, axis=-1, keepdims=True))
    e0out_ref[...] = e0 * jnp.minimum(1.0, 1.0 / jnp.maximum(en, 1e-7))

    p = lax.dot_general(u_n, rel_n, (((1,), (1,)), ((), ())))
    ids = rel_ref[...]
    s = jnp.zeros(ids.shape, jnp.float32)
    for r in range(32):
        s = s + jnp.where(ids == r, p[:, r:r + 1], 0.0)

    m = jnp.max(s, axis=1, keepdims=True)
    e = jnp.exp(s - m)
    w_ref[...] = e / jnp.sum(e, axis=1, keepdims=True)


def _tc_weights(rows_u, rows_e0, rel2, relation_table):
    bt = 512
    grid = B // bt
    return pl.pallas_call(
        _weights_body,
        grid=(grid,),
        in_specs=[
            pl.BlockSpec((bt, E_DIM), lambda i: (i, 0)),
            pl.BlockSpec((bt, E_DIM), lambda i: (i, 0)),
            pl.BlockSpec((bt, N_NEIGHBORS), lambda i: (i, 0)),
            pl.BlockSpec((32, E_DIM), lambda i: (0, 0)),
        ],
        out_specs=[
            pl.BlockSpec((bt, E_DIM), lambda i: (i, 0)),
            pl.BlockSpec((bt, E_DIM), lambda i: (i, 0)),
            pl.BlockSpec((bt, N_NEIGHBORS), lambda i: (i, 0)),
        ],
        out_shape=[
            jax.ShapeDtypeStruct((B, E_DIM), jnp.float32),
            jax.ShapeDtypeStruct((B, E_DIM), jnp.float32),
            jax.ShapeDtypeStruct((B, N_NEIGHBORS), jnp.float32),
        ],
    )(rows_u, rows_e0, rel2, relation_table)


def _sc2_body(nid_hbm, w_hbm, ent_tab_hbm, scale_hbm, out_agg,
              ids_v, w_v, sc_v, rows_v, agg_v, sem_a, sem_b):
    wid = lax.axis_index("s") * NC + lax.axis_index("c")

    def chunk_body(t, carry):
        base = wid * PER_W + t * CHUNK2
        h_ids = pltpu.async_copy(
            nid_hbm.at[pl.ds(base * 16, IDX2)], ids_v, sem_a)
        h_w = pltpu.async_copy(w_hbm.at[pl.ds(base * 16, IDX2)], w_v, sem_b)
        h_ids.wait()
        h_rows = []
        for j in range(N_STREAMS2):
            sl = pl.ds(j * STREAM, STREAM)
            h_rows.append(pltpu.async_copy(
                ent_tab_hbm.at[ids_v.at[sl]], rows_v.at[sl], sem_a))
            h_rows.append(pltpu.async_copy(
                scale_hbm.at[ids_v.at[sl]], sc_v.at[sl], sem_b))
        h_w.wait()
        for h in h_rows:
            h.wait()

        def elem_body(c, _):
            sl16 = pl.ds(pl.multiple_of(c * 16, 16), 16)
            wv = w_v[sl16] * sc_v[sl16]
            for g in range(E_DIM // 16):
                terms = [wv[k] * rows_v[c * 16 + k, pl.ds(g * 16, 16)]
                         for k in range(N_NEIGHBORS)]
                while len(terms) > 1:
                    terms = [terms[i] + terms[i + 1]
                             for i in range(0, len(terms), 2)]
                agg_v[c, pl.ds(g * 16, 16)] = terms[0]
            return 0

        lax.fori_loop(0, CHUNK2, elem_body, 0)

        pltpu.sync_copy(agg_v, out_agg.at[pl.ds(base, CHUNK2)])
        return carry

    lax.fori_loop(0, N_CHUNKS2, chunk_body, 0)


def _sc2_reduce(nbr_ids_flat, w_flat, entity_table, scales):
    mesh = plsc.VectorSubcoreMesh(core_axis_name="c", subcore_axis_name="s")
    f = functools.partial(
        pl.kernel,
        out_type=jax.ShapeDtypeStruct((B, E_DIM), jnp.float32),
        mesh=mesh,
        compiler_params=pltpu.CompilerParams(use_tc_tiling_on_sc=False),
        scratch_types=(
            pltpu.VMEM((IDX2,), jnp.int32),
            pltpu.VMEM((IDX2,), jnp.float32),
            pltpu.VMEM((IDX2,), jnp.float32),
            pltpu.VMEM((IDX2, E_DIM), jnp.float32),
            pltpu.VMEM((CHUNK2, E_DIM), jnp.float32),
            pltpu.SemaphoreType.DMA,
            pltpu.SemaphoreType.DMA,
        ),
    )(_sc2_body)
    return f(nbr_ids_flat, w_flat, entity_table, scales)


def _final_body(e0_ref, agg_ref, w_ref, b_ref, o_ref):
    out = (e0_ref[...] + agg_ref[...]) @ w_ref[...] + b_ref[...]
    o_ref[...] = jnp.tanh(out)


def _tc_final(e0_n, agg, W_agg, b2):
    bt = 1024
    grid = B // bt
    return pl.pallas_call(
        _final_body,
        grid=(grid,),
        in_specs=[
            pl.BlockSpec((bt, E_DIM), lambda i: (i, 0)),
            pl.BlockSpec((bt, E_DIM), lambda i: (i, 0)),
            pl.BlockSpec((E_DIM, E_DIM), lambda i: (0, 0)),
            pl.BlockSpec((1, E_DIM), lambda i: (0, 0)),
        ],
        out_specs=pl.BlockSpec((bt, E_DIM), lambda i: (i, 0)),
        out_shape=jax.ShapeDtypeStruct((B, E_DIM), jnp.float32),
    )(e0_n, agg, W_agg, b2)


def kernel(users, items, adj_entity, adj_relation, user_table, entity_table,
           relation_table, W_agg, b_agg):
    users_i = users.astype(jnp.int32)
    items_i = items.astype(jnp.int32)
    adj_e = adj_entity.astype(jnp.int32)
    adj_r = adj_relation.astype(jnp.int32)

    scales = _row_scales(entity_table).reshape(-1)
    rows_u, rows_e0, nbr_ids_flat, rel2 = _sc1_gather(
        users_i, items_i, adj_e, adj_r, user_table, entity_table)
    u_n, e0_n, w = _tc_weights(rows_u, rows_e0, rel2, relation_table)
    agg = _sc2_reduce(nbr_ids_flat, w.reshape(-1), entity_table, scales)
    item_out = _tc_final(e0_n, agg, W_agg, b_agg.reshape(1, E_DIM))
    return u_n.reshape(B, 1, E_DIM), item_out


# speedup vs baseline: 7.4536x; 1.2523x over previous
import functools

import jax
import jax.numpy as jnp
from jax import lax
from jax.experimental import pallas as pl
from jax.experimental.pallas import tpu as pltpu
from jax.experimental.pallas import tpu_sc as plsc

E_DIM = 64
N_NEIGHBORS = 16
B = 16384

NC = 2
NS = 16
NW = NC * NS
PER_W = B // NW

CHUNK1 = 64
N_CHUNKS1 = PER_W // CHUNK1
IDX1 = CHUNK1 * N_NEIGHBORS
CHUNK2 = 32
N_CHUNKS2 = PER_W // CHUNK2
IDX2 = CHUNK2 * N_NEIGHBORS
STREAM = 128
N_STREAMS2 = IDX2 // STREAM


def _norm_body(x_ref, o_ref):
    x = x_ref[...]
    n = jnp.sqrt(jnp.sum(x * x, axis=-1, keepdims=True))
    o_ref[...] = jnp.minimum(1.0, 1.0 / jnp.maximum(n, 1e-7))


def _row_scales(table):
    rows = table.shape[0]
    blk = 2000
    grid = rows // blk
    return pl.pallas_call(
        _norm_body,
        grid=(grid,),
        in_specs=[pl.BlockSpec((blk, E_DIM), lambda i: (i, 0))],
        out_specs=pl.BlockSpec((blk, 1), lambda i: (i, 0)),
        out_shape=jax.ShapeDtypeStruct((rows, 1), jnp.float32),
    )(table)


def _sc1_body(users_hbm, items_hbm, adj_e_hbm, adj_r_hbm, user_tab_hbm,
              ent_tab_hbm, out_u, out_e0, out_nid, out_rel,
              idx_u, idx_i, nbr_ids, nbr_flat, rel_buf, rows_u, rows_e0,
              sem_a, sem_b):
    wid = lax.axis_index("s") * NC + lax.axis_index("c")

    def chunk_body(t, carry):
        base = wid * PER_W + t * CHUNK1
        pltpu.sync_copy(users_hbm.at[pl.ds(base, CHUNK1)], idx_u)
        pltpu.sync_copy(items_hbm.at[pl.ds(base, CHUNK1)], idx_i)

        h_ids = pltpu.async_copy(adj_e_hbm.at[idx_i], nbr_ids, sem_a)
        h_rel = pltpu.async_copy(adj_r_hbm.at[idx_i], rel_buf, sem_b)
        h_u = pltpu.async_copy(user_tab_hbm.at[idx_u], rows_u, sem_b)
        h_e0 = pltpu.async_copy(ent_tab_hbm.at[idx_i], rows_e0, sem_b)

        h_ids.wait()

        def flat_body(c, _):
            v = nbr_ids[c, :]
            nbr_flat[pl.ds(pl.multiple_of(c * 16, 16), 16)] = v
            return 0

        lax.fori_loop(0, CHUNK1, flat_body, 0)

        h_rel.wait()
        h_u.wait()
        h_e0.wait()

        pltpu.sync_copy(rows_u, out_u.at[pl.ds(base, CHUNK1)])
        pltpu.sync_copy(rows_e0, out_e0.at[pl.ds(base, CHUNK1)])
        pltpu.sync_copy(nbr_flat, out_nid.at[pl.ds(base * 16, IDX1)])
        pltpu.sync_copy(rel_buf, out_rel.at[pl.ds(base, CHUNK1)])
        return carry

    lax.fori_loop(0, N_CHUNKS1, chunk_body, 0)


def _sc1_gather(users_i, items_i, adj_e, adj_r, user_table, entity_table):
    mesh = plsc.VectorSubcoreMesh(core_axis_name="c", subcore_axis_name="s")
    f = functools.partial(
        pl.kernel,
        out_type=(
            jax.ShapeDtypeStruct((B, E_DIM), jnp.float32),
            jax.ShapeDtypeStruct((B, E_DIM), jnp.float32),
            jax.ShapeDtypeStruct((B * N_NEIGHBORS,), jnp.int32),
            jax.ShapeDtypeStruct((B, N_NEIGHBORS), jnp.int32),
        ),
        mesh=mesh,
        compiler_params=pltpu.CompilerParams(use_tc_tiling_on_sc=False),
        scratch_types=(
            pltpu.VMEM((CHUNK1,), jnp.int32),
            pltpu.VMEM((CHUNK1,), jnp.int32),
            pltpu.VMEM((CHUNK1, N_NEIGHBORS), jnp.int32),
            pltpu.VMEM((IDX1,), jnp.int32),
            pltpu.VMEM((CHUNK1, N_NEIGHBORS), jnp.int32),
            pltpu.VMEM((CHUNK1, E_DIM), jnp.float32),
            pltpu.VMEM((CHUNK1, E_DIM), jnp.float32),
            pltpu.SemaphoreType.DMA,
            pltpu.SemaphoreType.DMA,
        ),
    )(_sc1_body)
    return f(users_i, items_i, adj_e, adj_r, user_table, entity_table)


def _weights_body(u_ref, e0_ref, rel_ref, reltab_ref, uout_ref, e0out_ref,
                  w_ref):
    rel = reltab_ref[...]
    rn = jnp.sqrt(jnp.sum(rel * rel, axis=-1, keepdims=True))
    rel_n = rel * jnp.minimum(1.0, 1.0 / jnp.maximum(rn, 1e-7))

    u = u_ref[...]
    un = jnp.sqrt(jnp.sum(u * u, axis=-1, keepdims=True))
    u_n = u * jnp.minimum(1.0, 1.0 / jnp.maximum(un, 1e-7))
    uout_ref[...] = u_n

    e0 = e0_ref[...]
    en = jnp.sqrt(jnp.sum(e0 * e0, axis=-1, keepdims=True))
    e0out_ref[...] = e0 * jnp.minimum(1.0, 1.0 / jnp.maximum(en, 1e-7))

    p = lax.dot_general(u_n, rel_n, (((1,), (1,)), ((), ())))
    ids = rel_ref[...]
    s = jnp.take_along_axis(p, ids, axis=1)

    m = jnp.max(s, axis=1, keepdims=True)
    e = jnp.exp(s - m)
    w_ref[...] = e / jnp.sum(e, axis=1, keepdims=True)


def _tc_weights(rows_u, rows_e0, rel2, relation_table):
    bt = 512
    grid = B // bt
    return pl.pallas_call(
        _weights_body,
        grid=(grid,),
        in_specs=[
            pl.BlockSpec((bt, E_DIM), lambda i: (i, 0)),
            pl.BlockSpec((bt, E_DIM), lambda i: (i, 0)),
            pl.BlockSpec((bt, N_NEIGHBORS), lambda i: (i, 0)),
            pl.BlockSpec((32, E_DIM), lambda i: (0, 0)),
        ],
        out_specs=[
            pl.BlockSpec((bt, E_DIM), lambda i: (i, 0)),
            pl.BlockSpec((bt, E_DIM), lambda i: (i, 0)),
            pl.BlockSpec((bt, N_NEIGHBORS), lambda i: (i, 0)),
        ],
        out_shape=[
            jax.ShapeDtypeStruct((B, E_DIM), jnp.float32),
            jax.ShapeDtypeStruct((B, E_DIM), jnp.float32),
            jax.ShapeDtypeStruct((B, N_NEIGHBORS), jnp.float32),
        ],
    )(rows_u, rows_e0, rel2, relation_table)


def _sc2_body(nid_hbm, w_hbm, ent_tab_hbm, scale_hbm, out_agg,
              ids0, ids1, w0, w1, sc0, sc1, rows0, rows1, agg_v,
              sem_r0, sem_r1, sem_s0, sem_s1):
    wid = lax.axis_index("s") * NC + lax.axis_index("c")
    ids_b = (ids0, ids1)
    w_b = (w0, w1)
    sc_b = (sc0, sc1)
    rows_b = (rows0, rows1)
    sem_r = (sem_r0, sem_r1)
    sem_s = (sem_s0, sem_s1)

    def fire(t, slot):
        base = wid * PER_W + t * CHUNK2
        pltpu.sync_copy(nid_hbm.at[pl.ds(base * 16, IDX2)], ids_b[slot])
        pltpu.sync_copy(w_hbm.at[pl.ds(base * 16, IDX2)], w_b[slot])
        for j in range(N_STREAMS2):
            sl = pl.ds(j * STREAM, STREAM)
            pltpu.async_copy(
                ent_tab_hbm.at[ids_b[slot].at[sl]], rows_b[slot].at[sl],
                sem_r[slot])
            pltpu.async_copy(
                scale_hbm.at[ids_b[slot].at[sl]], sc_b[slot].at[sl],
                sem_s[slot])

    def drain(slot):
        for j in range(N_STREAMS2):
            sl = pl.ds(j * STREAM, STREAM)
            pltpu.make_async_copy(
                ent_tab_hbm.at[ids_b[slot].at[sl]], rows_b[slot].at[sl],
                sem_r[slot]).wait()
            pltpu.make_async_copy(
                scale_hbm.at[ids_b[slot].at[sl]], sc_b[slot].at[sl],
                sem_s[slot]).wait()

    def compute(t, slot):
        base = wid * PER_W + t * CHUNK2
        w_v, sc_v, rows_v = w_b[slot], sc_b[slot], rows_b[slot]

        def elem_body(c, _):
            sl16 = pl.ds(pl.multiple_of(c * 16, 16), 16)
            wv = w_v[sl16] * sc_v[sl16]
            for g in range(E_DIM // 16):
                terms = [wv[k] * rows_v[c * 16 + k, pl.ds(g * 16, 16)]
                         for k in range(N_NEIGHBORS)]
                while len(terms) > 1:
                    terms = [terms[i] + terms[i + 1]
                             for i in range(0, len(terms), 2)]
                agg_v[c, pl.ds(g * 16, 16)] = terms[0]
            return 0

        lax.fori_loop(0, CHUNK2, elem_body, 0)
        pltpu.sync_copy(agg_v, out_agg.at[pl.ds(base, CHUNK2)])

    fire(0, 0)

    def pair_body(i, carry):
        t0 = 2 * i
        t1 = t0 + 1
        fire(t1, 1)
        drain(0)
        compute(t0, 0)

        @pl.when(t1 + 1 < N_CHUNKS2)
        def _():
            fire(t1 + 1, 0)

        drain(1)
        compute(t1, 1)
        return carry

    lax.fori_loop(0, N_CHUNKS2 // 2, pair_body, 0)


def _sc2_reduce(nbr_ids_flat, w_flat, entity_table, scales):
    mesh = plsc.VectorSubcoreMesh(core_axis_name="c", subcore_axis_name="s")
    f = functools.partial(
        pl.kernel,
        out_type=jax.ShapeDtypeStruct((B, E_DIM), jnp.float32),
        mesh=mesh,
        compiler_params=pltpu.CompilerParams(use_tc_tiling_on_sc=False),
        scratch_types=(
            pltpu.VMEM((IDX2,), jnp.int32),
            pltpu.VMEM((IDX2,), jnp.int32),
            pltpu.VMEM((IDX2,), jnp.float32),
            pltpu.VMEM((IDX2,), jnp.float32),
            pltpu.VMEM((IDX2,), jnp.float32),
            pltpu.VMEM((IDX2,), jnp.float32),
            pltpu.VMEM((IDX2, E_DIM), jnp.float32),
            pltpu.VMEM((IDX2, E_DIM), jnp.float32),
            pltpu.VMEM((CHUNK2, E_DIM), jnp.float32),
            pltpu.SemaphoreType.DMA,
            pltpu.SemaphoreType.DMA,
            pltpu.SemaphoreType.DMA,
            pltpu.SemaphoreType.DMA,
        ),
    )(_sc2_body)
    return f(nbr_ids_flat, w_flat, entity_table, scales)


def _final_body(e0_ref, agg_ref, w_ref, b_ref, o_ref):
    out = (e0_ref[...] + agg_ref[...]) @ w_ref[...] + b_ref[...]
    o_ref[...] = jnp.tanh(out)


def _tc_final(e0_n, agg, W_agg, b2):
    bt = 1024
    grid = B // bt
    return pl.pallas_call(
        _final_body,
        grid=(grid,),
        in_specs=[
            pl.BlockSpec((bt, E_DIM), lambda i: (i, 0)),
            pl.BlockSpec((bt, E_DIM), lambda i: (i, 0)),
            pl.BlockSpec((E_DIM, E_DIM), lambda i: (0, 0)),
            pl.BlockSpec((1, E_DIM), lambda i: (0, 0)),
        ],
        out_specs=pl.BlockSpec((bt, E_DIM), lambda i: (i, 0)),
        out_shape=jax.ShapeDtypeStruct((B, E_DIM), jnp.float32),
    )(e0_n, agg, W_agg, b2)


def kernel(users, items, adj_entity, adj_relation, user_table, entity_table,
           relation_table, W_agg, b_agg):
    users_i = users.astype(jnp.int32)
    items_i = items.astype(jnp.int32)
    adj_e = adj_entity.astype(jnp.int32)
    adj_r = adj_relation.astype(jnp.int32)

    scales = _row_scales(entity_table).reshape(-1)
    rows_u, rows_e0, nbr_ids_flat, rel2 = _sc1_gather(
        users_i, items_i, adj_e, adj_r, user_table, entity_table)
    u_n, e0_n, w = _tc_weights(rows_u, rows_e0, rel2, relation_table)
    agg = _sc2_reduce(nbr_ids_flat, w.reshape(-1), entity_table, scales)
    item_out = _tc_final(e0_n, agg, W_agg, b_agg.reshape(1, E_DIM))
    return u_n.reshape(B, 1, E_DIM), item_out
